# Initial kernel scaffold; baseline (speedup 1.0000x reference)
import jax
import jax.numpy as jnp
from jax.experimental import pallas as pl


def kernel(x):
    raise NotImplementedError("write your pallas kernel here")


# speedup vs baseline: 25.8957x; 25.8957x over previous
import jax
import jax.numpy as jnp
from jax.experimental import pallas as pl

_K = 512


def _tc_body(x_ref, o_ref):
    x = x_ref[...]
    r = jnp.maximum(x, 0.0)
    bits = jax.lax.bitcast_convert_type(r, jnp.int32)

    def step(i, t):
        b = 30 - i
        cand = t | jnp.left_shift(jnp.int32(1), b)
        cnt = jnp.sum((bits >= cand).astype(jnp.int32), axis=1, keepdims=True)
        return jnp.where(cnt >= _K, cand, t)

    t0 = jnp.zeros((x.shape[0], 1), jnp.int32)
    t = jax.lax.fori_loop(0, 31, step, t0)
    o_ref[...] = jnp.where(bits >= t, r, 0.0)


def kernel(x):
    return pl.pallas_call(
        _tc_body,
        out_shape=jax.ShapeDtypeStruct(x.shape, x.dtype),
    )(x)
